# baseline (speedup 1.0000x reference)
import functools

import jax
import jax.numpy as jnp
from jax import lax
from jax.experimental import pallas as pl
from jax.experimental.pallas import tpu as pltpu
from jax.experimental.pallas import tpu_sc as plsc

NUM_EMBEDDINGS = 1000000
EMBED_DIM = 32
NUM_FEATURES = 2
BATCH = 4096
MAX_LEN = 20

LANES = 16
PACK = 4
NUM_WORKERS = 32
BAGS_PER_WORKER = BATCH // NUM_WORKERS
IDX_PER_WORKER = BAGS_PER_WORKER * MAX_LEN
BAGS_PER_CHUNK = 8
ENT_PER_CHUNK = BAGS_PER_CHUNK * MAX_LEN
CHUNKS = BAGS_PER_WORKER // BAGS_PER_CHUNK


def _gather_chunk(table, idx_v, rows_buf, sem, start):
    c0 = pltpu.async_copy(
        table.at[idx_v.at[pl.ds(start, 128)]],
        rows_buf.at[pl.ds(0, 128)], sem)
    c1 = pltpu.async_copy(
        table.at[idx_v.at[pl.ds(start + 128, 32)]],
        rows_buf.at[pl.ds(128, 32)], sem)
    return (c0, c1)


def _compute_chunk(rows_buf, voff_v, acc_v, w, vbase, abase):
    accs = {}
    for g in range(ENT_PER_CHUNK // LANES):
        grp = voff_v[pl.ds(vbase + g * LANES, LANES)]
        for j in range(LANES):
            local = g * LANES + j
            bag = local // MAX_LEN
            l = local % MAX_LEN
            off = grp[j]
            x0 = rows_buf[local, pl.ds(off, LANES)]
            x1 = rows_buf[local, pl.ds(off + LANES, LANES)]
            if bag not in accs:
                accs[bag] = (w[l] * x0, w[l] * x1)
            else:
                a0, a1 = accs[bag]
                accs[bag] = (a0 + w[l] * x0, a1 + w[l] * x1)
            if l == MAX_LEN - 1:
                a0, a1 = accs.pop(bag)
                o = pl.multiple_of(abase + bag * EMBED_DIM, EMBED_DIM)
                acc_v[pl.ds(o, LANES)] = a0
                acc_v[pl.ds(o + LANES, LANES)] = a1


def _sc_body(idx4_hbm, voff_hbm, t0_hbm, t1_hbm, wv_hbm, out_hbm,
             idx_v, voff_v, rows0_v, rows1_v, acc_v, wv_v, sem0, sem1):
    cid = lax.axis_index("c")
    sid = lax.axis_index("s")
    wid = sid * 2 + cid
    base_bag = wid * BAGS_PER_WORKER

    pltpu.sync_copy(wv_hbm, wv_v)

    for f in range(NUM_FEATURES):
        table = t0_hbm if f == 0 else t1_hbm
        ebase = f * BATCH * MAX_LEN + wid * IDX_PER_WORKER

        pltpu.sync_copy(idx4_hbm.at[pl.ds(ebase, IDX_PER_WORKER)], idx_v)
        pltpu.sync_copy(voff_hbm.at[pl.ds(ebase, IDX_PER_WORKER)], voff_v)

        w = tuple(wv_v[pl.ds((f * MAX_LEN + l) * LANES, LANES)]
                  for l in range(MAX_LEN))

        _gather_chunk(table, idx_v, rows0_v, sem0, 0)

        def pair_body(p, w):
            c_even = p * 2
            c_odd = p * 2 + 1
            pltpu.make_async_copy(
                table.at[idx_v.at[pl.ds(0, 128)]],
                rows0_v.at[pl.ds(0, 128)], sem0).wait()
            pltpu.make_async_copy(
                table.at[idx_v.at[pl.ds(0, 32)]],
                rows0_v.at[pl.ds(128, 32)], sem0).wait()
            _gather_chunk(table, idx_v, rows1_v, sem1,
                          c_odd * ENT_PER_CHUNK)
            _compute_chunk(rows0_v, voff_v, acc_v, w,
                           c_even * ENT_PER_CHUNK,
                           c_even * BAGS_PER_CHUNK * EMBED_DIM)
            pltpu.make_async_copy(
                table.at[idx_v.at[pl.ds(0, 128)]],
                rows1_v.at[pl.ds(0, 128)], sem1).wait()
            pltpu.make_async_copy(
                table.at[idx_v.at[pl.ds(0, 32)]],
                rows1_v.at[pl.ds(128, 32)], sem1).wait()
            nxt = lax.min(c_even + 2, CHUNKS - 2)
            _gather_chunk(table, idx_v, rows0_v, sem0,
                          nxt * ENT_PER_CHUNK)
            _compute_chunk(rows1_v, voff_v, acc_v, w,
                           c_odd * ENT_PER_CHUNK,
                           c_odd * BAGS_PER_CHUNK * EMBED_DIM)
            return w

        lax.fori_loop(0, CHUNKS // 2, pair_body, w, unroll=False)

        pltpu.make_async_copy(
            table.at[idx_v.at[pl.ds(0, 128)]],
            rows0_v.at[pl.ds(0, 128)], sem0).wait()
        pltpu.make_async_copy(
            table.at[idx_v.at[pl.ds(0, 32)]],
            rows0_v.at[pl.ds(128, 32)], sem0).wait()

        pltpu.sync_copy(
            acc_v,
            out_hbm.at[pl.ds(f * BATCH * EMBED_DIM + base_bag * EMBED_DIM,
                             BAGS_PER_WORKER * EMBED_DIM)])


@jax.jit
def _fpebc(idx4, voff, rt0, rt1, wv):
    mesh = plsc.VectorSubcoreMesh(core_axis_name="c", subcore_axis_name="s")
    kern = functools.partial(
        pl.kernel,
        out_type=jax.ShapeDtypeStruct((NUM_FEATURES * BATCH * EMBED_DIM,),
                                      jnp.float32),
        mesh=mesh,
        compiler_params=pltpu.CompilerParams(use_tc_tiling_on_sc=True),
        scratch_types=[
            pltpu.VMEM((IDX_PER_WORKER,), jnp.int32),
            pltpu.VMEM((IDX_PER_WORKER,), jnp.int32),
            pltpu.VMEM((ENT_PER_CHUNK, PACK * EMBED_DIM), jnp.float32),
            pltpu.VMEM((ENT_PER_CHUNK, PACK * EMBED_DIM), jnp.float32),
            pltpu.VMEM((BAGS_PER_WORKER * EMBED_DIM,), jnp.float32),
            pltpu.VMEM((NUM_FEATURES * MAX_LEN * LANES,), jnp.float32),
            pltpu.SemaphoreType.DMA,
            pltpu.SemaphoreType.DMA,
        ],
    )(_sc_body)
    out_flat = kern(idx4, voff, rt0, rt1, wv)
    return (out_flat.reshape(NUM_FEATURES, BATCH, EMBED_DIM)
            .transpose(1, 0, 2)
            .reshape(BATCH, NUM_FEATURES * EMBED_DIM))


def kernel(indices, table0, table1, pos_w):
    rt0 = jnp.reshape(table0, (NUM_EMBEDDINGS // PACK, PACK * EMBED_DIM))
    rt1 = jnp.reshape(table1, (NUM_EMBEDDINGS // PACK, PACK * EMBED_DIM))
    idx_flat = indices.reshape(-1)
    idx4 = idx_flat // PACK
    voff = (idx_flat % PACK) * EMBED_DIM
    wv = jnp.broadcast_to(pos_w[:, :, None],
                          (NUM_FEATURES, MAX_LEN, LANES)).reshape(-1)
    return _fpebc(idx4, voff, rt0, rt1, wv)
